# baseline (speedup 1.0000x reference)
import functools

import jax
import jax.numpy as jnp
from jax import lax
from jax.experimental import pallas as pl
from jax.experimental.pallas import tpu as pltpu
from jax.experimental.pallas import tpu_sc as plsc

N = 10000
H = 128
E = 320000

NC = 2
NS = 16
NW = NC * NS
EW = E // NW
C = 40
NCHUNK = EW // C
RB = 7
RPT = 624
REM = N - NS * RPT

R = 1000
G = N // R


def _sc_aggregate(t, ei0, ei1):
    mesh = plsc.VectorSubcoreMesh(core_axis_name="c", subcore_axis_name="s")

    @functools.partial(
        pl.kernel,
        out_type=jax.ShapeDtypeStruct((2 * N, H), jnp.float32),
        mesh=mesh,
        scratch_types=(
            [pltpu.VMEM((EW,), jnp.int32)]
            + [pltpu.VMEM((C, H), jnp.float32)] * RB
            + [pltpu.VMEM((C,), jnp.int32)] * RB
            + [pltpu.VMEM_SHARED((N, H), jnp.float32)]
            + [pltpu.SemaphoreType.DMA] * (3 * RB)
        ),
    )
    def agg(t_hbm, ei0_hbm, ei1_hbm, out_hbm, *scr):
        eiA = scr[0]
        rows = scr[1:1 + RB]
        dst = scr[1 + RB:1 + 2 * RB]
        acc = scr[1 + 2 * RB]
        gsem = scr[2 + 2 * RB:2 + 3 * RB]
        ssem = scr[2 + 3 * RB:2 + 4 * RB]
        dsem = scr[2 + 4 * RB:2 + 5 * RB]

        c = lax.axis_index("c")
        s = lax.axis_index("s")
        wid = s * NC + c

        zeros16 = jnp.zeros((16,), jnp.float32)

        def zrow(r, carry):
            for j in range(H // 16):
                rows[0][r, pl.ds(j * 16, 16)] = zeros16
            return carry

        lax.fori_loop(0, C, zrow, 0)
        for k in range(RPT // C):
            pltpu.sync_copy(rows[0], acc.at[pl.ds(s * RPT + k * C, C)])
        pltpu.sync_copy(rows[0].at[pl.ds(0, RPT % C)],
                        acc.at[pl.ds(s * RPT + (RPT // C) * C, RPT % C)])

        @pl.when(s == NS - 1)
        def _zero_rem():
            pltpu.sync_copy(rows[0].at[pl.ds(0, REM)],
                            acc.at[pl.ds(NS * RPT, REM)])

        plsc.subcore_barrier()

        def gstart(g, b):
            pltpu.async_copy(
                t_hbm.at[eiA.at[pl.ds(g * C, C)]], rows[b], gsem[b])

        def gwait(g, b):
            pltpu.make_async_copy(
                t_hbm.at[eiA.at[pl.ds(g * C, C)]], rows[b], gsem[b]).wait()

        def sstart(b):
            pltpu.async_copy(rows[b], acc.at[dst[b]], ssem[b], add=True)

        def swait(b):
            pltpu.make_async_copy(rows[b], acc.at[dst[b]], ssem[b]).wait()

        for (src_hbm, dst_hbm) in ((ei0_hbm, ei1_hbm), (ei1_hbm, ei0_hbm)):
            pltpu.sync_copy(src_hbm.at[pl.ds(wid * EW, EW)], eiA)

            def dstart(g, b, dst_hbm=dst_hbm):
                pltpu.async_copy(
                    dst_hbm.at[pl.ds(wid * EW + g * C, C)], dst[b], dsem[b])

            def dwait(g, b, dst_hbm=dst_hbm):
                pltpu.make_async_copy(
                    dst_hbm.at[pl.ds(wid * EW + g * C, C)], dst[b],
                    dsem[b]).wait()

            def process(g, b, wait_prev=True):
                dwait(g, b)
                gwait(g, b)
                if wait_prev:
                    swait((b + RB - 1) % RB)
                sstart(b)

            def launch(g, b):
                dstart(g, b)
                gstart(g, b)

            for k in range(RB - 1):
                launch(k, k)
            for k in range(RB - 1):
                process(k, k, wait_prev=(k > 0))
                launch(k + RB - 1, (k + RB - 1) % RB)

            def body(i, carry):
                g = (RB - 1) + RB * i
                for b in range(RB):
                    k = g + b
                    process(k, (RB - 1 + b) % RB)
                    launch(k + RB - 1, (2 * (RB - 1) + b) % RB)
                return carry

            lax.fori_loop(0, (NCHUNK - 2 * (RB - 1)) // RB, body, 0)
            for k in range(NCHUNK - (RB - 1), NCHUNK):
                process(k, k % RB)
            swait((NCHUNK - 1) % RB)

        plsc.subcore_barrier()
        pltpu.sync_copy(acc.at[pl.ds(s * RPT, RPT)],
                        out_hbm.at[pl.ds(c * N + s * RPT, RPT)])

        @pl.when(s == NS - 1)
        def _out_rem():
            pltpu.sync_copy(acc.at[pl.ds(NS * RPT, REM)],
                            out_hbm.at[pl.ds(c * N + NS * RPT, REM)])

    return agg(t, ei0, ei1)


def _tc_encode(x, We, be, W0, b0):
    D = x.shape[1]

    def body(x_ref, we_ref, be_ref, w0_ref, b0_ref, x0_ref, t_ref):
        x0 = jnp.dot(x_ref[...], we_ref[...],
                     preferred_element_type=jnp.float32) + be_ref[...]
        x0_ref[...] = x0
        t_ref[...] = jnp.dot(x0, w0_ref[...],
                             preferred_element_type=jnp.float32) + b0_ref[...]

    return pl.pallas_call(
        body,
        grid=(G,),
        in_specs=[
            pl.BlockSpec((R, D), lambda i: (i, 0)),
            pl.BlockSpec((D, H), lambda i: (0, 0)),
            pl.BlockSpec((1, H), lambda i: (0, 0)),
            pl.BlockSpec((H, H), lambda i: (0, 0)),
            pl.BlockSpec((1, H), lambda i: (0, 0)),
        ],
        out_specs=(
            pl.BlockSpec((R, H), lambda i: (i, 0)),
            pl.BlockSpec((R, H), lambda i: (i, 0)),
        ),
        out_shape=(
            jax.ShapeDtypeStruct((N, H), jnp.float32),
            jax.ShapeDtypeStruct((N, H), jnp.float32),
        ),
    )(x, We, be.reshape(1, H), W0, b0.reshape(1, H))


def _tc_mid(P, x0, W, b):
    def body(p0_ref, p1_ref, x0_ref, w_ref, b_ref, t_ref):
        h = jnp.maximum(p0_ref[...] + p1_ref[...] + x0_ref[...], 0.0)
        t_ref[...] = jnp.dot(h, w_ref[...],
                             preferred_element_type=jnp.float32) + b_ref[...]

    return pl.pallas_call(
        body,
        grid=(G,),
        in_specs=[
            pl.BlockSpec((R, H), lambda i: (i, 0)),
            pl.BlockSpec((R, H), lambda i: (i + G, 0)),
            pl.BlockSpec((R, H), lambda i: (i, 0)),
            pl.BlockSpec((H, H), lambda i: (0, 0)),
            pl.BlockSpec((1, H), lambda i: (0, 0)),
        ],
        out_specs=pl.BlockSpec((R, H), lambda i: (i, 0)),
        out_shape=jax.ShapeDtypeStruct((N, H), jnp.float32),
    )(P, P, x0, W, b.reshape(1, H))


def _tc_final(P, x0, Wd, bd):
    def body(p0_ref, p1_ref, x0_ref, wd_ref, bd_ref, o_ref):
        h = jnp.maximum(p0_ref[...] + p1_ref[...] + x0_ref[...], 0.0)
        o_ref[...] = jnp.sum(h * wd_ref[...], axis=1,
                             keepdims=True) + bd_ref[...]

    return pl.pallas_call(
        body,
        grid=(G,),
        in_specs=[
            pl.BlockSpec((R, H), lambda i: (i, 0)),
            pl.BlockSpec((R, H), lambda i: (i + G, 0)),
            pl.BlockSpec((R, H), lambda i: (i, 0)),
            pl.BlockSpec((1, H), lambda i: (0, 0)),
            pl.BlockSpec((1, 1), lambda i: (0, 0)),
        ],
        out_specs=pl.BlockSpec((R, 1), lambda i: (i, 0)),
        out_shape=jax.ShapeDtypeStruct((N, 1), jnp.float32),
    )(P, P, x0, Wd.reshape(1, H), bd.reshape(1, 1))


def kernel(x, edge_index, W_enc, b_enc, W_layers, b_layers, W_dec, b_dec):
    L = W_layers.shape[0]
    ei0 = edge_index[0]
    ei1 = edge_index[1]
    x0, t = _tc_encode(x, W_enc, b_enc, W_layers[0], b_layers[0])
    out = None
    for i in range(L):
        P = _sc_aggregate(t, ei0, ei1)
        if i + 1 < L:
            t = _tc_mid(P, x0, W_layers[i + 1], b_layers[i + 1])
        else:
            out = _tc_final(P, x0, W_dec, b_dec)
    return out


# speedup vs baseline: 1.0008x; 1.0008x over previous
import functools

import jax
import jax.numpy as jnp
from jax import lax
from jax.experimental import pallas as pl
from jax.experimental.pallas import tpu as pltpu
from jax.experimental.pallas import tpu_sc as plsc

N = 10000
H = 128
E = 320000

NC = 2
NS = 16
NW = NC * NS
EW = E // NW
C = 40
NCHUNK = EW // C
RB = 6
RPT = 624
REM = N - NS * RPT

R = 1000
G = N // R


def _sc_aggregate(t, zeros, ei0, ei1):
    mesh = plsc.VectorSubcoreMesh(core_axis_name="c", subcore_axis_name="s")

    @functools.partial(
        pl.kernel,
        out_type=jax.ShapeDtypeStruct((2 * N, H), jnp.float32),
        mesh=mesh,
        scratch_types=(
            [pltpu.VMEM((EW,), jnp.int32)]
            + [pltpu.VMEM((C, H), jnp.float32)] * RB
            + [pltpu.VMEM((C,), jnp.int32)] * RB
            + [pltpu.VMEM_SHARED((N, H), jnp.float32)]
            + [pltpu.SemaphoreType.DMA] * (3 * RB)
            + [pltpu.SemaphoreType.DMA]
        ),
    )
    def agg(t_hbm, z_hbm, ei0_hbm, ei1_hbm, out_hbm, *scr):
        eiA = scr[0]
        rows = scr[1:1 + RB]
        dst = scr[1 + RB:1 + 2 * RB]
        acc = scr[1 + 2 * RB]
        gsem = scr[2 + 2 * RB:2 + 3 * RB]
        ssem = scr[2 + 3 * RB:2 + 4 * RB]
        dsem = scr[2 + 4 * RB:2 + 5 * RB]
        zsem = scr[2 + 5 * RB]

        c = lax.axis_index("c")
        s = lax.axis_index("s")
        wid = s * NC + c

        pltpu.async_copy(z_hbm.at[pl.ds(s * RPT, RPT)],
                         acc.at[pl.ds(s * RPT, RPT)], zsem)

        @pl.when(s == NS - 1)
        def _zero_rem():
            pltpu.async_copy(z_hbm.at[pl.ds(NS * RPT, REM)],
                             acc.at[pl.ds(NS * RPT, REM)], zsem)

        def zwait():
            pltpu.make_async_copy(z_hbm.at[pl.ds(s * RPT, RPT)],
                                  acc.at[pl.ds(s * RPT, RPT)], zsem).wait()

            @pl.when(s == NS - 1)
            def _zwait_rem():
                pltpu.make_async_copy(z_hbm.at[pl.ds(NS * RPT, REM)],
                                      acc.at[pl.ds(NS * RPT, REM)],
                                      zsem).wait()

        def gstart(g, b):
            pltpu.async_copy(
                t_hbm.at[eiA.at[pl.ds(g * C, C)]], rows[b], gsem[b])

        def gwait(g, b):
            pltpu.make_async_copy(
                t_hbm.at[eiA.at[pl.ds(g * C, C)]], rows[b], gsem[b]).wait()

        def sstart(b):
            pltpu.async_copy(rows[b], acc.at[dst[b]], ssem[b], add=True)

        def swait(b):
            pltpu.make_async_copy(rows[b], acc.at[dst[b]], ssem[b]).wait()

        for di, (src_hbm, dst_hbm) in enumerate(
                ((ei0_hbm, ei1_hbm), (ei1_hbm, ei0_hbm))):
            pltpu.sync_copy(src_hbm.at[pl.ds(wid * EW, EW)], eiA)

            def dstart(g, b, dst_hbm=dst_hbm):
                pltpu.async_copy(
                    dst_hbm.at[pl.ds(wid * EW + g * C, C)], dst[b], dsem[b])

            def dwait(g, b, dst_hbm=dst_hbm):
                pltpu.make_async_copy(
                    dst_hbm.at[pl.ds(wid * EW + g * C, C)], dst[b],
                    dsem[b]).wait()

            def process(g, b, wait_prev=True):
                dwait(g, b)
                gwait(g, b)
                if wait_prev:
                    swait((b + RB - 1) % RB)
                sstart(b)

            def launch(g, b):
                dstart(g, b)
                gstart(g, b)

            for k in range(RB - 1):
                launch(k, k)
            if di == 0:
                zwait()
                plsc.subcore_barrier()
            for k in range(RB - 1):
                process(k, k, wait_prev=(k > 0))
                launch(k + RB - 1, (k + RB - 1) % RB)

            def body(i, carry):
                g = (RB - 1) + RB * i
                for b in range(RB):
                    k = g + b
                    process(k, (RB - 1 + b) % RB)
                    launch(k + RB - 1, (2 * (RB - 1) + b) % RB)
                return carry

            lax.fori_loop(0, (NCHUNK - 2 * (RB - 1)) // RB, body, 0)
            for k in range(NCHUNK - (RB - 1), NCHUNK):
                process(k, k % RB)
            swait((NCHUNK - 1) % RB)

        plsc.subcore_barrier()
        pltpu.sync_copy(acc.at[pl.ds(s * RPT, RPT)],
                        out_hbm.at[pl.ds(c * N + s * RPT, RPT)])

        @pl.when(s == NS - 1)
        def _out_rem():
            pltpu.sync_copy(acc.at[pl.ds(NS * RPT, REM)],
                            out_hbm.at[pl.ds(c * N + NS * RPT, REM)])

    return agg(t, zeros, ei0, ei1)


def _tc_encode(x, We, be, W0, b0):
    D = x.shape[1]

    def body(x_ref, we_ref, be_ref, w0_ref, b0_ref, x0_ref, t_ref):
        x0 = jnp.dot(x_ref[...], we_ref[...],
                     preferred_element_type=jnp.float32) + be_ref[...]
        x0_ref[...] = x0
        t_ref[...] = jnp.dot(x0, w0_ref[...],
                             preferred_element_type=jnp.float32) + b0_ref[...]

    return pl.pallas_call(
        body,
        grid=(G,),
        in_specs=[
            pl.BlockSpec((R, D), lambda i: (i, 0)),
            pl.BlockSpec((D, H), lambda i: (0, 0)),
            pl.BlockSpec((1, H), lambda i: (0, 0)),
            pl.BlockSpec((H, H), lambda i: (0, 0)),
            pl.BlockSpec((1, H), lambda i: (0, 0)),
        ],
        out_specs=(
            pl.BlockSpec((R, H), lambda i: (i, 0)),
            pl.BlockSpec((R, H), lambda i: (i, 0)),
        ),
        out_shape=(
            jax.ShapeDtypeStruct((N, H), jnp.float32),
            jax.ShapeDtypeStruct((N, H), jnp.float32),
        ),
    )(x, We, be.reshape(1, H), W0, b0.reshape(1, H))


def _tc_mid(P, x0, W, b):
    def body(p0_ref, p1_ref, x0_ref, w_ref, b_ref, t_ref):
        h = jnp.maximum(p0_ref[...] + p1_ref[...] + x0_ref[...], 0.0)
        t_ref[...] = jnp.dot(h, w_ref[...],
                             preferred_element_type=jnp.float32) + b_ref[...]

    return pl.pallas_call(
        body,
        grid=(G,),
        in_specs=[
            pl.BlockSpec((R, H), lambda i: (i, 0)),
            pl.BlockSpec((R, H), lambda i: (i + G, 0)),
            pl.BlockSpec((R, H), lambda i: (i, 0)),
            pl.BlockSpec((H, H), lambda i: (0, 0)),
            pl.BlockSpec((1, H), lambda i: (0, 0)),
        ],
        out_specs=pl.BlockSpec((R, H), lambda i: (i, 0)),
        out_shape=jax.ShapeDtypeStruct((N, H), jnp.float32),
    )(P, P, x0, W, b.reshape(1, H))


def _tc_final(P, x0, Wd, bd):
    def body(p0_ref, p1_ref, x0_ref, wd_ref, bd_ref, o_ref):
        h = jnp.maximum(p0_ref[...] + p1_ref[...] + x0_ref[...], 0.0)
        o_ref[...] = jnp.sum(h * wd_ref[...], axis=1,
                             keepdims=True) + bd_ref[...]

    return pl.pallas_call(
        body,
        grid=(G,),
        in_specs=[
            pl.BlockSpec((R, H), lambda i: (i, 0)),
            pl.BlockSpec((R, H), lambda i: (i + G, 0)),
            pl.BlockSpec((R, H), lambda i: (i, 0)),
            pl.BlockSpec((1, H), lambda i: (0, 0)),
            pl.BlockSpec((1, 1), lambda i: (0, 0)),
        ],
        out_specs=pl.BlockSpec((R, 1), lambda i: (i, 0)),
        out_shape=jax.ShapeDtypeStruct((N, 1), jnp.float32),
    )(P, P, x0, Wd.reshape(1, H), bd.reshape(1, 1))


def kernel(x, edge_index, W_enc, b_enc, W_layers, b_layers, W_dec, b_dec):
    L = W_layers.shape[0]
    ei0 = edge_index[0]
    ei1 = edge_index[1]
    zeros = jnp.zeros((N, H), jnp.float32)
    x0, t = _tc_encode(x, W_enc, b_enc, W_layers[0], b_layers[0])
    out = None
    for i in range(L):
        P = _sc_aggregate(t, zeros, ei0, ei1)
        if i + 1 < L:
            t = _tc_mid(P, x0, W_layers[i + 1], b_layers[i + 1])
        else:
            out = _tc_final(P, x0, W_dec, b_dec)
    return out
